# baseline (speedup 1.0000x reference)
import jax
import jax.numpy as jnp
from jax.experimental import pallas as pl
from jax.experimental.pallas import tpu as pltpu

N_ROWS = 4096
N_CLASSES = 401
MAX_ATTRS = 16
BLOCK_ROWS = 512
N_BLOCKS = N_ROWS // BLOCK_ROWS
CHUNK = 64


def _chunk_loss(x, ids):
    lane = jax.lax.broadcasted_iota(jnp.int32, (CHUNK, N_CLASSES), 1)
    m = lane == ids[:, 0:1]
    for j in range(1, MAX_ATTRS):
        m = m | (lane == ids[:, j:j + 1])
    mf = m.astype(jnp.float32)

    mx = jnp.max(x, axis=1, keepdims=True)
    se = jnp.sum(jnp.exp(x - mx), axis=1, keepdims=True)
    lse = mx + jnp.log(se)
    gp = jnp.sum(mf * x, axis=1, keepdims=True)
    dp = jnp.sum(mf, axis=1, keepdims=True)
    nzf = (ids != 0).astype(jnp.float32)
    cnt = jnp.sum(nzf, axis=1, keepdims=True)
    any0 = (cnt < MAX_ATTRS).astype(jnp.float32)
    g = gp - any0 * x[:, 0:1]
    d = dp - any0
    row_loss = (d * lse - g) / jnp.maximum(cnt, 1.0)
    return jnp.sum(row_loss)


def _copies(lref, aref, xbuf, abuf, sx, sa, block, slot):
    r0 = block * BLOCK_ROWS
    return (
        pltpu.make_async_copy(lref.at[pl.ds(r0, BLOCK_ROWS), :],
                              xbuf.at[slot], sx.at[slot]),
        pltpu.make_async_copy(aref.at[pl.ds(r0, BLOCK_ROWS), :],
                              abuf.at[slot], sa.at[slot]),
    )


def _kernel_body(lref, aref, out_ref, xbuf, abuf, sx, sa):
    i = pl.program_id(0)
    slot = jax.lax.rem(i, 2)
    nxt = jax.lax.rem(i + 1, 2)

    @pl.when(i == 0)
    def _():
        out_ref[...] = jnp.zeros((1, 1), jnp.float32)
        for cp in _copies(lref, aref, xbuf, abuf, sx, sa, 0, 0):
            cp.start()

    @pl.when(i + 1 < N_BLOCKS)
    def _():
        for cp in _copies(lref, aref, xbuf, abuf, sx, sa, i + 1, nxt):
            cp.start()

    for cp in _copies(lref, aref, xbuf, abuf, sx, sa, i, slot):
        cp.wait()

    s = jnp.zeros((), jnp.float32)
    for c in range(BLOCK_ROWS // CHUNK):
        s = s + _chunk_loss(xbuf[slot, pl.ds(c * CHUNK, CHUNK), :],
                            abuf[slot, pl.ds(c * CHUNK, CHUNK), :])
    out_ref[...] += s.reshape(1, 1) * (1.0 / N_ROWS)


def kernel(attribute_logits, attributes):
    out = pl.pallas_call(
        _kernel_body,
        grid=(N_BLOCKS,),
        in_specs=[
            pl.BlockSpec(memory_space=pl.ANY),
            pl.BlockSpec(memory_space=pl.ANY),
        ],
        out_specs=pl.BlockSpec((1, 1), lambda i: (0, 0)),
        out_shape=jax.ShapeDtypeStruct((1, 1), jnp.float32),
        scratch_shapes=[
            pltpu.VMEM((2, BLOCK_ROWS, N_CLASSES), jnp.float32),
            pltpu.VMEM((2, BLOCK_ROWS, MAX_ATTRS), jnp.int32),
            pltpu.SemaphoreType.DMA((2,)),
            pltpu.SemaphoreType.DMA((2,)),
        ],
    )(attribute_logits, attributes)
    return out[0, 0]
